# baseline (device time: 149574 ns/iter reference)
import jax
import jax.numpy as jnp
from jax import lax
from jax.experimental import pallas as pl
from jax.experimental.pallas import tpu as pltpu

N_DEV = 16
M = 1024
N = 1024
CH = M // N_DEV


def _gelu(z):
    return 0.5 * z * (1.0 + jnp.tanh(0.7978845608 * (z + 0.044715 * z * z * z)))


def kernel(A, B):
    m, k = A.shape
    _, n = B.shape

    def body(a_ref, b_ref, out_ref, z_ref, comm_ref,
             rs_ssem, rs_rsem, ag_ssem, ag_rsem):
        d = lax.axis_index("i")
        left = lax.rem(d + N_DEV - 1, N_DEV)
        right = lax.rem(d + 1, N_DEV)

        barrier_sem = pltpu.get_barrier_semaphore()
        for nbr in (left, right):
            pl.semaphore_signal(
                barrier_sem, inc=1,
                device_id=(nbr,), device_id_type=pl.DeviceIdType.MESH,
            )
        pl.semaphore_wait(barrier_sem, 2)

        z_ref[:, :] = jnp.dot(
            a_ref[:, :], b_ref[:, :], preferred_element_type=jnp.float32
        )

        comm_ref[0, :, :] = z_ref[pl.ds(d * CH, CH), :]
        for h in range(N_DEV - 1):
            rdma = pltpu.make_async_remote_copy(
                src_ref=comm_ref.at[h],
                dst_ref=comm_ref.at[h + 1],
                send_sem=rs_ssem.at[h],
                recv_sem=rs_rsem.at[h],
                device_id=(right,),
                device_id_type=pl.DeviceIdType.MESH,
            )
            rdma.start()
            rdma.wait()
            c = lax.rem(d + 2 * N_DEV - 1 - h, N_DEV)
            comm_ref[h + 1, :, :] += z_ref[pl.ds(c * CH, CH), :]

        own = lax.rem(d + 1, N_DEV)
        out_ref[pl.ds(own * CH, CH), :] = _gelu(comm_ref[N_DEV - 1, :, :])

        for g in range(N_DEV - 1):
            x = lax.rem(own + 2 * N_DEV - g, N_DEV)
            rdma = pltpu.make_async_remote_copy(
                src_ref=out_ref.at[pl.ds(x * CH, CH), :],
                dst_ref=out_ref.at[pl.ds(x * CH, CH), :],
                send_sem=ag_ssem.at[g],
                recv_sem=ag_rsem.at[g],
                device_id=(right,),
                device_id_type=pl.DeviceIdType.MESH,
            )
            rdma.start()
            rdma.wait()

    return pl.pallas_call(
        body,
        out_shape=jax.ShapeDtypeStruct((M, N), jnp.float32),
        in_specs=[
            pl.BlockSpec(memory_space=pltpu.VMEM),
            pl.BlockSpec(memory_space=pltpu.VMEM),
        ],
        out_specs=pl.BlockSpec(memory_space=pltpu.VMEM),
        scratch_shapes=[
            pltpu.VMEM((m, n), jnp.float32),
            pltpu.VMEM((N_DEV, CH, n), jnp.float32),
            pltpu.SemaphoreType.DMA((N_DEV - 1,)),
            pltpu.SemaphoreType.DMA((N_DEV - 1,)),
            pltpu.SemaphoreType.DMA((N_DEV - 1,)),
            pltpu.SemaphoreType.DMA((N_DEV - 1,)),
        ],
        compiler_params=pltpu.CompilerParams(collective_id=0),
    )(A, B)


# device time: 146097 ns/iter; 1.0238x vs baseline; 1.0238x over previous
import jax
import jax.numpy as jnp
from jax import lax
from jax.experimental import pallas as pl
from jax.experimental.pallas import tpu as pltpu

N_DEV = 16
M = 1024
N = 1024
CH = M // N_DEV
H = CH // 2


def _gelu(z):
    return 0.5 * z * (1.0 + jnp.tanh(0.7978845608 * (z + 0.044715 * z * z * z)))


def kernel(A, B):
    m, k = A.shape
    _, n = B.shape

    def body(a_ref, b_ref, out_ref, z_ref, cw_ref, ccw_ref,
             cw_ssem, cw_rsem, ccw_ssem, ccw_rsem):
        d = lax.axis_index("i")
        left = lax.rem(d + N_DEV - 1, N_DEV)
        right = lax.rem(d + 1, N_DEV)

        barrier_sem = pltpu.get_barrier_semaphore()
        for nbr in (left, right):
            pl.semaphore_signal(
                barrier_sem, inc=1,
                device_id=(nbr,), device_id_type=pl.DeviceIdType.MESH,
            )
        pl.semaphore_wait(barrier_sem, 2)

        def rs_pair(h):
            cw = pltpu.make_async_remote_copy(
                src_ref=cw_ref.at[h],
                dst_ref=cw_ref.at[h + 1],
                send_sem=cw_ssem.at[h],
                recv_sem=cw_rsem.at[h],
                device_id=(right,),
                device_id_type=pl.DeviceIdType.MESH,
            )
            ccw = pltpu.make_async_remote_copy(
                src_ref=ccw_ref.at[h],
                dst_ref=ccw_ref.at[h + 1],
                send_sem=ccw_ssem.at[h],
                recv_sem=ccw_rsem.at[h],
                device_id=(left,),
                device_id_type=pl.DeviceIdType.MESH,
            )
            return cw, ccw

        z_ref[pl.ds(d * CH, CH), :] = jnp.dot(
            a_ref[pl.ds(d * CH, CH), :], b_ref[:, :],
            preferred_element_type=jnp.float32,
        )
        cw_ref[0, :, :] = z_ref[pl.ds(d * CH, H), :]
        ccw_ref[0, :, :] = z_ref[pl.ds(d * CH + H, H), :]
        cw0, ccw0 = rs_pair(0)
        cw0.start()
        ccw0.start()

        z_ref[:, :] = jnp.dot(
            a_ref[:, :], b_ref[:, :], preferred_element_type=jnp.float32
        )

        for h in range(N_DEV - 1):
            if h == 0:
                cw, ccw = cw0, ccw0
            else:
                cw, ccw = rs_pair(h)
                cw.start()
                ccw.start()
            cw.wait()
            ccw.wait()
            c_cw = lax.rem(d + 2 * N_DEV - 1 - h, N_DEV)
            c_ccw = lax.rem(d + 1 + h, N_DEV)
            cw_ref[h + 1, :, :] += z_ref[pl.ds(c_cw * CH, H), :]
            ccw_ref[h + 1, :, :] += z_ref[pl.ds(c_ccw * CH + H, H), :]

        own_cw = lax.rem(d + 1, N_DEV)
        own_ccw = lax.rem(d + N_DEV - 1, N_DEV)
        out_ref[pl.ds(own_cw * CH, H), :] = _gelu(cw_ref[N_DEV - 1, :, :])
        out_ref[pl.ds(own_ccw * CH + H, H), :] = _gelu(ccw_ref[N_DEV - 1, :, :])

        for g in range(N_DEV - 1):
            x = lax.rem(own_cw + 2 * N_DEV - g, N_DEV)
            y = lax.rem(own_ccw + g, N_DEV)
            cw = pltpu.make_async_remote_copy(
                src_ref=out_ref.at[pl.ds(x * CH, H), :],
                dst_ref=out_ref.at[pl.ds(x * CH, H), :],
                send_sem=cw_ssem.at[g],
                recv_sem=cw_rsem.at[g],
                device_id=(right,),
                device_id_type=pl.DeviceIdType.MESH,
            )
            ccw = pltpu.make_async_remote_copy(
                src_ref=out_ref.at[pl.ds(y * CH + H, H), :],
                dst_ref=out_ref.at[pl.ds(y * CH + H, H), :],
                send_sem=ccw_ssem.at[g],
                recv_sem=ccw_rsem.at[g],
                device_id=(left,),
                device_id_type=pl.DeviceIdType.MESH,
            )
            cw.start()
            ccw.start()
            cw.wait()
            ccw.wait()

    return pl.pallas_call(
        body,
        out_shape=jax.ShapeDtypeStruct((M, N), jnp.float32),
        in_specs=[
            pl.BlockSpec(memory_space=pltpu.VMEM),
            pl.BlockSpec(memory_space=pltpu.VMEM),
        ],
        out_specs=pl.BlockSpec(memory_space=pltpu.VMEM),
        scratch_shapes=[
            pltpu.VMEM((m, n), jnp.float32),
            pltpu.VMEM((N_DEV, H, n), jnp.float32),
            pltpu.VMEM((N_DEV, H, n), jnp.float32),
            pltpu.SemaphoreType.DMA((N_DEV - 1,)),
            pltpu.SemaphoreType.DMA((N_DEV - 1,)),
            pltpu.SemaphoreType.DMA((N_DEV - 1,)),
            pltpu.SemaphoreType.DMA((N_DEV - 1,)),
        ],
        compiler_params=pltpu.CompilerParams(collective_id=0),
    )(A, B)


# device time: 78808 ns/iter; 1.8980x vs baseline; 1.8538x over previous
import jax
import jax.numpy as jnp
from jax import lax
from jax.experimental import pallas as pl
from jax.experimental.pallas import tpu as pltpu

N_DEV = 16
M = 1024
N = 1024

_MESH = pl.DeviceIdType.MESH

SZ = [256, 128, 64, 32]
ROFF = [0, 256, 384, 448]
AGSZ = [32, 64, 128, 256]


def _gelu(z):
    return 0.5 * z * (1.0 + jnp.tanh(0.7978845608 * (z + 0.044715 * z * z * z)))


def kernel(A, B):
    m, k = A.shape
    _, n = B.shape

    def body(a_ref, b_ref, out_ref, z_ref, recv_ref,
             rs_ssem, rs_rsem, ag_ssem, ag_rsem):
        d = lax.axis_index("i")
        q = lax.rem(d, 4)
        zc = lax.div(d, 4)
        i32 = jnp.int32
        qx = jnp.where((q == 1) | (q == 2), i32(1), i32(0))
        qy = jnp.where(q >= 2, i32(1), i32(0))
        zb0 = lax.rem(zc, 2)
        zb1 = lax.div(zc, 2)
        pid_x = 4 * zc + (q ^ 1)
        pid_y = 4 * zc + (q ^ 3)
        pid_z1 = 4 * (zc ^ 1) + q
        pid_z2 = 4 * (zc ^ 2) + q

        barrier_sem = pltpu.get_barrier_semaphore()
        for p in (pid_x, pid_y, pid_z1, pid_z2):
            pl.semaphore_signal(
                barrier_sem, inc=1, device_id=(p,), device_id_type=_MESH,
            )
        pl.semaphore_wait(barrier_sem, 4)

        partners = [[pid_x, pid_y, pid_z1, pid_z2],
                    [pid_y, pid_x, pid_z1, pid_z2]]
        bits = [[qx, qy, zb0, zb1], [qy, qx, zb0, zb1]]
        base = [0, 512]
        roff = [0, 480]

        sends = []

        for i in range(2):
            s0 = base[i] + (1 - bits[i][0]) * 256
            z_ref[pl.ds(s0, 256), :] = jnp.dot(
                a_ref[pl.ds(s0, 256), :], b_ref[:, :],
                preferred_element_type=jnp.float32,
            )

        descs = []
        for i in range(2):
            s0 = base[i] + (1 - bits[i][0]) * 256
            rd = pltpu.make_async_remote_copy(
                src_ref=z_ref.at[pl.ds(s0, 256), :],
                dst_ref=recv_ref.at[pl.ds(roff[i] + ROFF[0], 256), :],
                send_sem=rs_ssem.at[i],
                recv_sem=rs_rsem.at[i],
                device_id=(partners[i][0],),
                device_id_type=_MESH,
            )
            rd.start()
            descs.append(rd)

        for i in range(2):
            k0 = base[i] + bits[i][0] * 256
            z_ref[pl.ds(k0, 256), :] = jnp.dot(
                a_ref[pl.ds(k0, 256), :], b_ref[:, :],
                preferred_element_type=jnp.float32,
            )

        cur = [base[0], base[1]]
        for rnd in range(4):
            if rnd > 0:
                descs = []
                for i in range(2):
                    send_s = cur[i] + (1 - bits[i][rnd]) * SZ[rnd]
                    rd = pltpu.make_async_remote_copy(
                        src_ref=z_ref.at[pl.ds(send_s, SZ[rnd]), :],
                        dst_ref=recv_ref.at[
                            pl.ds(roff[i] + ROFF[rnd], SZ[rnd]), :],
                        send_sem=rs_ssem.at[2 * rnd + i],
                        recv_sem=rs_rsem.at[2 * rnd + i],
                        device_id=(partners[i][rnd],),
                        device_id_type=_MESH,
                    )
                    rd.start()
                    descs.append(rd)
            for i in range(2):
                descs[i].wait_recv()
                keep_s = cur[i] + bits[i][rnd] * SZ[rnd]
                z_ref[pl.ds(keep_s, SZ[rnd]), :] += recv_ref[
                    pl.ds(roff[i] + ROFF[rnd], SZ[rnd]), :]
                cur[i] = keep_s
            sends += descs

        for i in range(2):
            out_ref[pl.ds(cur[i], 32), :] = _gelu(z_ref[pl.ds(cur[i], 32), :])

        ag_partners = [[pid_z2, pid_z1, pid_y, pid_x],
                       [pid_z2, pid_z1, pid_x, pid_y]]
        ag_bits = [[zb1, zb0, qy, qx], [zb1, zb0, qx, qy]]
        for g in range(4):
            descs = []
            for i in range(2):
                rd = pltpu.make_async_remote_copy(
                    src_ref=out_ref.at[pl.ds(cur[i], AGSZ[g]), :],
                    dst_ref=out_ref.at[pl.ds(cur[i], AGSZ[g]), :],
                    send_sem=ag_ssem.at[2 * g + i],
                    recv_sem=ag_rsem.at[2 * g + i],
                    device_id=(ag_partners[i][g],),
                    device_id_type=_MESH,
                )
                rd.start()
                descs.append(rd)
            for i in range(2):
                descs[i].wait_recv()
                cur[i] = cur[i] - ag_bits[i][g] * AGSZ[g]
            sends += descs

        for rd in sends:
            rd.wait_send()

    return pl.pallas_call(
        body,
        out_shape=jax.ShapeDtypeStruct((M, N), jnp.float32),
        in_specs=[
            pl.BlockSpec(memory_space=pltpu.VMEM),
            pl.BlockSpec(memory_space=pltpu.VMEM),
        ],
        out_specs=pl.BlockSpec(memory_space=pltpu.VMEM),
        scratch_shapes=[
            pltpu.VMEM((m, n), jnp.float32),
            pltpu.VMEM((960, n), jnp.float32),
            pltpu.SemaphoreType.DMA((8,)),
            pltpu.SemaphoreType.DMA((8,)),
            pltpu.SemaphoreType.DMA((8,)),
            pltpu.SemaphoreType.DMA((8,)),
        ],
        compiler_params=pltpu.CompilerParams(collective_id=0),
    )(A, B)


# device time: 64383 ns/iter; 2.3232x vs baseline; 1.2240x over previous
import jax
import jax.numpy as jnp
from jax import lax
from jax.experimental import pallas as pl
from jax.experimental.pallas import tpu as pltpu

N_DEV = 16
M = 1024
N = 1024
HC = 512

_MESH = pl.DeviceIdType.MESH

SZ = [256, 128, 64, 32]
ROFF = [0, 256, 384, 448]
AGSZ = [32, 64, 128, 256]


def _gelu(z):
    return 0.5 * z * (1.0 + jnp.tanh(0.7978845608 * (z + 0.044715 * z * z * z)))


def kernel(A, B):
    m, k = A.shape
    _, n = B.shape

    def body(a_ref, b_ref, out_ref, z_ref, recv_ref,
             rs_ssem, rs_rsem, ag_ssem, ag_rsem):
        d = lax.axis_index("i")
        q = lax.rem(d, 4)
        zc = lax.div(d, 4)
        i32 = jnp.int32
        qx = jnp.where((q == 1) | (q == 2), i32(1), i32(0))
        qy = jnp.where(q >= 2, i32(1), i32(0))
        zb0 = lax.rem(zc, 2)
        zb1 = lax.div(zc, 2)
        pid_x = 4 * zc + (q ^ 1)
        pid_y = 4 * zc + (q ^ 3)
        pid_z1 = 4 * (zc ^ 1) + q
        pid_z2 = 4 * (zc ^ 2) + q

        barrier_sem = pltpu.get_barrier_semaphore()
        for p in (pid_x, pid_y, pid_z1, pid_z2):
            pl.semaphore_signal(
                barrier_sem, inc=1, device_id=(p,), device_id_type=_MESH,
            )
        pl.semaphore_wait(barrier_sem, 4)

        rail_partners = [[pid_x, pid_y, pid_z1, pid_z2],
                         [pid_y, pid_x, pid_z1, pid_z2]]
        rail_bits = [[qx, qy, zb0, zb1], [qy, qx, zb0, zb1]]
        ag_partners = [[pid_z2, pid_z1, pid_y, pid_x],
                       [pid_z2, pid_z1, pid_x, pid_y]]
        ag_bits = [[zb1, zb0, qy, qx], [zb1, zb0, qx, qy]]
        rail_base = [0, 512]
        rail_roff = [0, 480]
        streams = [(0, 0), (1, 0), (0, 1), (1, 1)]
        NS = len(streams)

        def colslice(s):
            ch = streams[s][1]
            return slice(ch * HC, (ch + 1) * HC)

        def rs_desc(s, rnd, cur_s):
            rail = streams[s][0]
            send_s = cur_s + (1 - rail_bits[rail][rnd]) * SZ[rnd]
            return pltpu.make_async_remote_copy(
                src_ref=z_ref.at[pl.ds(send_s, SZ[rnd]), colslice(s)],
                dst_ref=recv_ref.at[
                    pl.ds(rail_roff[rail] + ROFF[rnd], SZ[rnd]), colslice(s)],
                send_sem=rs_ssem.at[NS * rnd + s],
                recv_sem=rs_rsem.at[NS * rnd + s],
                device_id=(rail_partners[rail][rnd],),
                device_id_type=_MESH,
            )

        def ag_desc(s, g, cur_s):
            rail = streams[s][0]
            return pltpu.make_async_remote_copy(
                src_ref=out_ref.at[pl.ds(cur_s, AGSZ[g]), colslice(s)],
                dst_ref=out_ref.at[pl.ds(cur_s, AGSZ[g]), colslice(s)],
                send_sem=ag_ssem.at[NS * g + s],
                recv_sem=ag_rsem.at[NS * g + s],
                device_id=(ag_partners[rail][g],),
                device_id_type=_MESH,
            )

        sends = []

        for i in range(2):
            s0 = rail_base[i] + (1 - rail_bits[i][0]) * 256
            z_ref[pl.ds(s0, 256), :] = jnp.dot(
                a_ref[pl.ds(s0, 256), :], b_ref[:, :],
                preferred_element_type=jnp.float32,
            )

        cur = [rail_base[streams[s][0]] for s in range(NS)]
        live = {}
        for s in range(NS):
            rd = rs_desc(s, 0, cur[s])
            rd.start()
            live[s] = rd

        for i in range(2):
            k0 = rail_base[i] + rail_bits[i][0] * 256
            z_ref[pl.ds(k0, 256), :] = jnp.dot(
                a_ref[pl.ds(k0, 256), :], b_ref[:, :],
                preferred_element_type=jnp.float32,
            )

        for rnd in range(4):
            nxt = {}
            for s in range(NS):
                rail = streams[s][0]
                live[s].wait_recv()
                sends.append(live[s])
                keep_s = cur[s] + rail_bits[rail][rnd] * SZ[rnd]
                z_ref[pl.ds(keep_s, SZ[rnd]), colslice(s)] += recv_ref[
                    pl.ds(rail_roff[rail] + ROFF[rnd], SZ[rnd]), colslice(s)]
                cur[s] = keep_s
                if rnd < 3:
                    nd = rs_desc(s, rnd + 1, cur[s])
                else:
                    out_ref[pl.ds(cur[s], 32), colslice(s)] = _gelu(
                        z_ref[pl.ds(cur[s], 32), colslice(s)])
                    nd = ag_desc(s, 0, cur[s])
                nd.start()
                nxt[s] = nd
            live = nxt

        for g in range(4):
            nxt = {}
            for s in range(NS):
                rail = streams[s][0]
                live[s].wait_recv()
                sends.append(live[s])
                cur[s] = cur[s] - ag_bits[rail][g] * AGSZ[g]
                if g < 3:
                    nd = ag_desc(s, g + 1, cur[s])
                    nd.start()
                    nxt[s] = nd
            live = nxt

        for rd in sends:
            rd.wait_send()

    return pl.pallas_call(
        body,
        out_shape=jax.ShapeDtypeStruct((M, N), jnp.float32),
        in_specs=[
            pl.BlockSpec(memory_space=pltpu.VMEM),
            pl.BlockSpec(memory_space=pltpu.VMEM),
        ],
        out_specs=pl.BlockSpec(memory_space=pltpu.VMEM),
        scratch_shapes=[
            pltpu.VMEM((m, n), jnp.float32),
            pltpu.VMEM((960, n), jnp.float32),
            pltpu.SemaphoreType.DMA((16,)),
            pltpu.SemaphoreType.DMA((16,)),
            pltpu.SemaphoreType.DMA((16,)),
            pltpu.SemaphoreType.DMA((16,)),
        ],
        compiler_params=pltpu.CompilerParams(collective_id=0),
    )(A, B)


# device time: 61891 ns/iter; 2.4167x vs baseline; 1.0403x over previous
import jax
import jax.numpy as jnp
from jax import lax
from jax.experimental import pallas as pl
from jax.experimental.pallas import tpu as pltpu

N_DEV = 16
M = 1024
N = 1024
QC = 256

_MESH = pl.DeviceIdType.MESH

SZ = [256, 128, 64, 32]
ROFF = [0, 256, 384, 448]
AGSZ = [32, 64, 128, 256]


def _gelu(z):
    return 0.5 * z * (1.0 + jnp.tanh(0.7978845608 * (z + 0.044715 * z * z * z)))


def kernel(A, B):
    m, k = A.shape
    _, n = B.shape

    def body(a_ref, b_ref, out_ref, z_ref, recv_ref,
             rs_ssem, rs_rsem, ag_ssem, ag_rsem):
        d = lax.axis_index("i")
        q = lax.rem(d, 4)
        zc = lax.div(d, 4)
        i32 = jnp.int32
        qx = jnp.where((q == 1) | (q == 2), i32(1), i32(0))
        qy = jnp.where(q >= 2, i32(1), i32(0))
        zb0 = lax.rem(zc, 2)
        zb1 = lax.div(zc, 2)
        pid_x = 4 * zc + (q ^ 1)
        pid_y = 4 * zc + (q ^ 3)
        pid_z1 = 4 * (zc ^ 1) + q
        pid_z2 = 4 * (zc ^ 2) + q

        barrier_sem = pltpu.get_barrier_semaphore()
        for p in (pid_x, pid_y, pid_z1, pid_z2):
            pl.semaphore_signal(
                barrier_sem, inc=1, device_id=(p,), device_id_type=_MESH,
            )
        pl.semaphore_wait(barrier_sem, 4)

        rail_partners = [[pid_x, pid_y, pid_z1, pid_z2],
                         [pid_y, pid_x, pid_z1, pid_z2]]
        rail_bits = [[qx, qy, zb0, zb1], [qy, qx, zb0, zb1]]
        ag_partners = [[pid_z2, pid_z1, pid_y, pid_x],
                       [pid_z2, pid_z1, pid_x, pid_y]]
        ag_bits = [[zb1, zb0, qy, qx], [zb1, zb0, qx, qy]]
        rail_base = [0, 512]
        rail_roff = [0, 480]
        streams = [(0, 0), (1, 0), (0, 1), (1, 1),
                   (0, 2), (1, 2), (0, 3), (1, 3)]
        NS = len(streams)

        def colslice(s):
            ch = streams[s][1]
            return slice(ch * QC, (ch + 1) * QC)

        def rs_desc(s, rnd, cur_s):
            rail = streams[s][0]
            send_s = cur_s + (1 - rail_bits[rail][rnd]) * SZ[rnd]
            return pltpu.make_async_remote_copy(
                src_ref=z_ref.at[pl.ds(send_s, SZ[rnd]), colslice(s)],
                dst_ref=recv_ref.at[
                    pl.ds(rail_roff[rail] + ROFF[rnd], SZ[rnd]), colslice(s)],
                send_sem=rs_ssem.at[NS * rnd + s],
                recv_sem=rs_rsem.at[NS * rnd + s],
                device_id=(rail_partners[rail][rnd],),
                device_id_type=_MESH,
            )

        def ag_desc(s, g, cur_s):
            rail = streams[s][0]
            return pltpu.make_async_remote_copy(
                src_ref=out_ref.at[pl.ds(cur_s, AGSZ[g]), colslice(s)],
                dst_ref=out_ref.at[pl.ds(cur_s, AGSZ[g]), colslice(s)],
                send_sem=ag_ssem.at[NS * g + s],
                recv_sem=ag_rsem.at[NS * g + s],
                device_id=(ag_partners[rail][g],),
                device_id_type=_MESH,
            )

        sends = []

        cur = [rail_base[streams[s][0]] for s in range(NS)]
        live = {}
        for s in range(NS):
            rail = streams[s][0]
            s0 = rail_base[rail] + (1 - rail_bits[rail][0]) * 256
            z_ref[pl.ds(s0, 256), colslice(s)] = jnp.dot(
                a_ref[pl.ds(s0, 256), :], b_ref[:, colslice(s)],
                preferred_element_type=jnp.float32,
            )
            rd = rs_desc(s, 0, cur[s])
            rd.start()
            live[s] = rd

        for i in range(2):
            k0 = rail_base[i] + rail_bits[i][0] * 256
            z_ref[pl.ds(k0, 256), :] = jnp.dot(
                a_ref[pl.ds(k0, 256), :], b_ref[:, :],
                preferred_element_type=jnp.float32,
            )

        for rnd in range(4):
            nxt = {}
            for s in range(NS):
                rail = streams[s][0]
                live[s].wait_recv()
                sends.append(live[s])
                keep_s = cur[s] + rail_bits[rail][rnd] * SZ[rnd]
                z_ref[pl.ds(keep_s, SZ[rnd]), colslice(s)] += recv_ref[
                    pl.ds(rail_roff[rail] + ROFF[rnd], SZ[rnd]), colslice(s)]
                cur[s] = keep_s
                if rnd < 3:
                    nd = rs_desc(s, rnd + 1, cur[s])
                else:
                    out_ref[pl.ds(cur[s], 32), colslice(s)] = _gelu(
                        z_ref[pl.ds(cur[s], 32), colslice(s)])
                    nd = ag_desc(s, 0, cur[s])
                nd.start()
                nxt[s] = nd
            live = nxt

        for g in range(4):
            nxt = {}
            for s in range(NS):
                rail = streams[s][0]
                live[s].wait_recv()
                sends.append(live[s])
                cur[s] = cur[s] - ag_bits[rail][g] * AGSZ[g]
                if g < 3:
                    nd = ag_desc(s, g + 1, cur[s])
                    nd.start()
                    nxt[s] = nd
            live = nxt

        for rd in sends:
            rd.wait_send()

    return pl.pallas_call(
        body,
        out_shape=jax.ShapeDtypeStruct((M, N), jnp.float32),
        in_specs=[
            pl.BlockSpec(memory_space=pltpu.VMEM),
            pl.BlockSpec(memory_space=pltpu.VMEM),
        ],
        out_specs=pl.BlockSpec(memory_space=pltpu.VMEM),
        scratch_shapes=[
            pltpu.VMEM((m, n), jnp.float32),
            pltpu.VMEM((960, n), jnp.float32),
            pltpu.SemaphoreType.DMA((32,)),
            pltpu.SemaphoreType.DMA((32,)),
            pltpu.SemaphoreType.DMA((32,)),
            pltpu.SemaphoreType.DMA((32,)),
        ],
        compiler_params=pltpu.CompilerParams(collective_id=0),
    )(A, B)


# device time: 50118 ns/iter; 2.9844x vs baseline; 1.2349x over previous
import jax
import jax.numpy as jnp
from jax import lax
from jax.experimental import pallas as pl
from jax.experimental.pallas import tpu as pltpu

N_DEV = 16
M = 1024
N = 1024
QC = 256

_MESH = pl.DeviceIdType.MESH

RAIL_BASE = [0, 384, 768]
RAIL_ROWS = [384, 384, 256]
RSZ = [[r // 2, r // 4, r // 8, r // 16] for r in RAIL_ROWS]
RROFF = [[0, r // 2, 3 * r // 4, 7 * r // 8] for r in RAIL_ROWS]
RAIL_ROFF = [0, 360, 720]


def _gelu(z):
    return 0.5 * z * (1.0 + jnp.tanh(0.7978845608 * (z + 0.044715 * z * z * z)))


def kernel(A, B):
    m, k = A.shape
    _, n = B.shape

    def body(a_ref, b_ref, out_ref, z_ref, recv_ref,
             rs_ssem, rs_rsem, ag_ssem, ag_rsem):
        d = lax.axis_index("i")
        q = lax.rem(d, 4)
        zc = lax.div(d, 4)
        i32 = jnp.int32
        qx = jnp.where((q == 1) | (q == 2), i32(1), i32(0))
        qy = jnp.where(q >= 2, i32(1), i32(0))
        zb0 = lax.rem(zc, 2)
        zb1 = lax.div(zc, 2)
        pid_x = 4 * zc + (q ^ 1)
        pid_y = 4 * zc + (q ^ 3)
        pid_z1 = 4 * (zc ^ 1) + q
        pid_z2 = 4 * (zc ^ 2) + q

        barrier_sem = pltpu.get_barrier_semaphore()
        for p in (pid_x, pid_y, pid_z1, pid_z2):
            pl.semaphore_signal(
                barrier_sem, inc=1, device_id=(p,), device_id_type=_MESH,
            )
        pl.semaphore_wait(barrier_sem, 4)

        streams = [(r, c) for c in range(4) for r in range(3)]
        NS = len(streams)
        _DIMS = {
            0: [(pid_x, qx), (pid_y, qy), (pid_z1, zb0), (pid_z2, zb1)],
            1: [(pid_y, qy), (pid_x, qx), (pid_z1, zb0), (pid_z2, zb1)],
            2: [(pid_z1, zb0), (pid_z2, zb1), (pid_x, qx), (pid_y, qy)],
        }

        def stream_dims(s):
            return _DIMS[streams[s][0]]

        def colslice(s):
            ch = streams[s][1]
            return slice(ch * QC, (ch + 1) * QC)

        def rs_desc(s, rnd, cur_s):
            rail = streams[s][0]
            _, bit = stream_dims(s)[rnd]
            sz = RSZ[rail][rnd]
            send_s = cur_s + (1 - bit) * sz
            return pltpu.make_async_remote_copy(
                src_ref=z_ref.at[pl.ds(send_s, sz), colslice(s)],
                dst_ref=recv_ref.at[
                    pl.ds(RAIL_ROFF[rail] + RROFF[rail][rnd], sz), colslice(s)],
                send_sem=rs_ssem.at[NS * rnd + s],
                recv_sem=rs_rsem.at[NS * rnd + s],
                device_id=(stream_dims(s)[rnd][0],),
                device_id_type=_MESH,
            )

        def ag_piece(s, slot, start_s, nrows, g):
            partner, _ = stream_dims(s)[3 - g]
            return pltpu.make_async_remote_copy(
                src_ref=out_ref.at[pl.ds(start_s, nrows), colslice(s)],
                dst_ref=out_ref.at[pl.ds(start_s, nrows), colslice(s)],
                send_sem=ag_ssem.at[NS * slot + s],
                recv_sem=ag_rsem.at[NS * slot + s],
                device_id=(partner,),
                device_id_type=_MESH,
            )

        sends = []

        cur = [RAIL_BASE[streams[s][0]] for s in range(NS)]
        live = {}
        for s in range(NS):
            rail = streams[s][0]
            h0 = RSZ[rail][0]
            s0 = RAIL_BASE[rail] + (1 - stream_dims(s)[0][1]) * h0
            z_ref[pl.ds(s0, h0), colslice(s)] = jnp.dot(
                a_ref[pl.ds(s0, h0), :], b_ref[:, colslice(s)],
                preferred_element_type=jnp.float32,
            )
            rd = rs_desc(s, 0, cur[s])
            rd.start()
            live[s] = rd

        for i in range(3):
            h0 = RSZ[i][0]
            k0 = RAIL_BASE[i] + [qx, qy, zb0][i] * h0
            z_ref[pl.ds(k0, h0), :] = jnp.dot(
                a_ref[pl.ds(k0, h0), :], b_ref[:, :],
                preferred_element_type=jnp.float32,
            )

        pend = {s: {} for s in range(NS)}
        for rnd in range(4):
            nxt = {}
            for s in range(NS):
                rail = streams[s][0]
                sz = RSZ[rail][rnd]
                live[s].wait_recv()
                sends.append(live[s])
                keep_s = cur[s] + stream_dims(s)[rnd][1] * sz
                z_ref[pl.ds(keep_s, sz), colslice(s)] += recv_ref[
                    pl.ds(RAIL_ROFF[rail] + RROFF[rail][rnd], sz), colslice(s)]
                cur[s] = keep_s
                if rnd < 3:
                    nd = rs_desc(s, rnd + 1, cur[s])
                    nd.start()
                    nxt[s] = nd
                else:
                    own = RSZ[rail][3]
                    out_ref[pl.ds(cur[s], own), colslice(s)] = _gelu(
                        z_ref[pl.ds(cur[s], own), colslice(s)])
                    for slot, g in ((0, 0), (1, 1)):
                        pc = ag_piece(s, slot, cur[s], own, g)
                        pc.start()
                        pend[s][slot] = pc
            live = nxt

        for g in range(4):
            for s in range(NS):
                rail = streams[s][0]
                agsz = RSZ[rail][3 - g]
                bit = stream_dims(s)[3 - g][1]
                slots = (0,) if g == 0 else (2 * g - 1, 2 * g)
                for sl in slots:
                    pend[s][sl].wait_recv()
                    sends.append(pend[s][sl])
                newcur = cur[s] - bit * agsz
                recv_start = newcur + (1 - bit) * agsz
                cur[s] = newcur
                if g < 3:
                    pc = ag_piece(s, 2 * g + 2, recv_start, agsz, g + 1)
                    pc.start()
                    pend[s][2 * g + 2] = pc
                if g < 2:
                    pc = ag_piece(s, 2 * g + 3, cur[s], RSZ[rail][2 - g],
                                  g + 2)
                    pc.start()
                    pend[s][2 * g + 3] = pc

        for rd in sends:
            rd.wait_send()

    return pl.pallas_call(
        body,
        out_shape=jax.ShapeDtypeStruct((M, N), jnp.float32),
        in_specs=[
            pl.BlockSpec(memory_space=pltpu.VMEM),
            pl.BlockSpec(memory_space=pltpu.VMEM),
        ],
        out_specs=pl.BlockSpec(memory_space=pltpu.VMEM),
        scratch_shapes=[
            pltpu.VMEM((m, n), jnp.float32),
            pltpu.VMEM((960, n), jnp.float32),
            pltpu.SemaphoreType.DMA((48,)),
            pltpu.SemaphoreType.DMA((48,)),
            pltpu.SemaphoreType.DMA((84,)),
            pltpu.SemaphoreType.DMA((84,)),
        ],
        compiler_params=pltpu.CompilerParams(collective_id=0),
    )(A, B)
